# Initial kernel scaffold; baseline (speedup 1.0000x reference)
import jax
import jax.numpy as jnp
from jax.experimental import pallas as pl


def kernel(query, value, W_off, b_off, W_attn, b_attn, W_val, b_val, W_out, b_out, spatial_shape):
    raise NotImplementedError("write your pallas kernel here")


# speedup vs baseline: 86.0328x; 86.0328x over previous
import functools

import jax
import jax.numpy as jnp
from jax import lax
from jax.experimental import pallas as pl
from jax.experimental.pallas import tpu as pltpu
from jax.experimental.pallas import tpu_sc as plsc

_B, _NQ, _D = 16, 1024, 256
_H, _W, _NH, _NP = 32, 32, 8, 4
_HD = _D // _NH
_NPC = _NP * 4
_NC, _NS = 2, 16
_NWORK = _NC * _NS
_PAIRS_PER_W = (_B * _NH) // _NWORK


def _prep_body(qt_ref, vtin_ref, wval_ref, bval_ref, woff_ref, boff_ref,
               wattn_ref, battn_ref, vt_ref, idx_ref, wgt_ref):
    qt = qt_ref[0]
    vin = vtin_ref[0]

    vt_ref[0] = (jnp.dot(wval_ref[...], vin, preferred_element_type=jnp.float32)
                 + bval_ref[...])

    offr = (jnp.dot(woff_ref[...], qt, preferred_element_type=jnp.float32)
            + boff_ref[...])
    off = jnp.tanh(offr)
    awr = (jnp.dot(wattn_ref[...], qt, preferred_element_type=jnp.float32)
           + battn_ref[...])

    aws = [awr[p * _NH:(p + 1) * _NH] for p in range(_NP)]
    m = jnp.maximum(jnp.maximum(aws[0], aws[1]), jnp.maximum(aws[2], aws[3]))
    es = [jnp.exp(a - m) for a in aws]
    rs = 1.0 / (es[0] + es[1] + es[2] + es[3])

    qi = lax.broadcasted_iota(jnp.int32, (_NH, _NQ), 1)
    gx = (qi % _W).astype(jnp.float32) * (2.0 / (_W - 1)) - 1.0
    gy = (qi // _W).astype(jnp.float32) * (2.0 / (_H - 1)) - 1.0

    for p in range(_NP):
        offx = off[p * _NH:(p + 1) * _NH]
        offy = off[32 + p * _NH:32 + (p + 1) * _NH]
        awn = es[p] * rs
        locx = jnp.clip(gx + 0.5 * offx, -1.0, 1.0)
        locy = jnp.clip(gy + 0.5 * offy, -1.0, 1.0)
        x = (locx + 1.0) * (_W / 2.0) - 0.5
        y = (locy + 1.0) * (_H / 2.0) - 0.5
        x0f = jnp.floor(x)
        y0f = jnp.floor(y)
        wx1 = x - x0f
        wy1 = y - y0f
        ix0 = x0f.astype(jnp.int32)
        iy0 = y0f.astype(jnp.int32)
        for c, (cy, cx) in enumerate(((0, 0), (0, 1), (1, 0), (1, 1))):
            ix = ix0 + cx
            iy = iy0 + cy
            wx = wx1 if cx else 1.0 - wx1
            wy = wy1 if cy else 1.0 - wy1
            valid = ((ix >= 0) & (ix <= _W - 1) & (iy >= 0) & (iy <= _H - 1))
            idxc = jnp.clip(iy, 0, _H - 1) * _W + jnp.clip(ix, 0, _W - 1)
            wc = wx * wy * awn * valid.astype(jnp.float32)
            pc = c * _NP + p
            idx_ref[0, :, pc, :] = idxc
            wgt_ref[0, :, pc, :] = wc


def _out_body(st_ref, wout_ref, bout_ref, o_ref):
    o_ref[0] = (lax.dot_general(st_ref[0], wout_ref[...],
                                (((0,), (1,)), ((), ())),
                                preferred_element_type=jnp.float32)
                + bout_ref[...])


def _sc_body(vt_hbm, idx_hbm, wgt_hbm, out_hbm, table, idxs, wgts, outv):
    wid = lax.axis_index("c") * _NS + lax.axis_index("s")

    def pair_body(k, carry):
        e = wid * _PAIRS_PER_W + k
        b = e // _NH
        h = e - b * _NH
        toff = pl.multiple_of(h * (_HD * _NQ), 8)
        ioff = pl.multiple_of(h * (_NPC * _NQ), 8)
        pltpu.sync_copy(vt_hbm.at[b, pl.ds(toff, _HD * _NQ)], table)
        pltpu.sync_copy(idx_hbm.at[b, pl.ds(ioff, _NPC * _NQ)], idxs)
        pltpu.sync_copy(wgt_hbm.at[b, pl.ds(ioff, _NPC * _NQ)], wgts)

        def q_body(qb, qcarry):
            q0 = pl.multiple_of(qb * 16, 16)
            for half in range(2):
                accs = [jnp.zeros((16,), jnp.float32) for _ in range(16)]
                for pc in range(_NPC):
                    st = pl.multiple_of(pc * _NQ, 16) + q0
                    rows = idxs[pl.ds(st, 16)]
                    wv = wgts[pl.ds(st, 16)]
                    for dd in range(16):
                        d = half * 16 + dd
                        g = plsc.load_gather(table, [rows + d * _NQ])
                        accs[dd] = accs[dd] + wv * g
                for dd in range(16):
                    d = half * 16 + dd
                    outv[pl.ds(pl.multiple_of(d * _NQ, 16) + q0, 16)] = accs[dd]
            return qcarry

        lax.fori_loop(0, _NQ // 16, q_body, 0)
        pltpu.sync_copy(outv, out_hbm.at[b, pl.ds(toff, _HD * _NQ)])
        return carry

    lax.fori_loop(0, _PAIRS_PER_W, pair_body, 0)


def _sc_gather(vt_flat, idx_flat, wgt_flat, *, interpret=False):
    mesh = plsc.VectorSubcoreMesh(core_axis_name="c", subcore_axis_name="s",
                                  num_cores=_NC, num_subcores=_NS)
    return pl.kernel(
        _sc_body,
        out_type=jax.ShapeDtypeStruct((_B, _HD * _NQ * _NH), jnp.float32),
        mesh=mesh,
        scratch_types=[
            pltpu.VMEM((_HD * _NQ,), jnp.float32),
            pltpu.VMEM((_NPC * _NQ,), jnp.int32),
            pltpu.VMEM((_NPC * _NQ,), jnp.float32),
            pltpu.VMEM((_HD * _NQ,), jnp.float32),
        ],
        compiler_params=pltpu.CompilerParams(needs_layout_passes=False),
        interpret=interpret,
    )(vt_flat, idx_flat, wgt_flat)


def _prep_call(qt, vtin, W_val, b_val_c, W_off_r, b_off_r, W_attn_r, b_attn_r,
               *, interpret=False):
    full = lambda shape: pl.BlockSpec(shape, lambda b: (0,) * len(shape))
    return pl.pallas_call(
        _prep_body,
        grid=(_B,),
        in_specs=[
            pl.BlockSpec((1, _D, _NQ), lambda b: (b, 0, 0)),
            pl.BlockSpec((1, _D, _NQ), lambda b: (b, 0, 0)),
            full((_D, _D)),
            full((_D, 1)),
            full((2 * _NP * _NH, _D)),
            full((2 * _NP * _NH, 1)),
            full((_NP * _NH, _D)),
            full((_NP * _NH, 1)),
        ],
        out_specs=[
            pl.BlockSpec((1, _D, _NQ), lambda b: (b, 0, 0)),
            pl.BlockSpec((1, _NH, _NPC, _NQ), lambda b: (b, 0, 0, 0)),
            pl.BlockSpec((1, _NH, _NPC, _NQ), lambda b: (b, 0, 0, 0)),
        ],
        out_shape=[
            jax.ShapeDtypeStruct((_B, _D, _NQ), jnp.float32),
            jax.ShapeDtypeStruct((_B, _NH, _NPC, _NQ), jnp.int32),
            jax.ShapeDtypeStruct((_B, _NH, _NPC, _NQ), jnp.float32),
        ],
        interpret=interpret,
    )(qt, vtin, W_val, b_val_c, W_off_r, b_off_r, W_attn_r, b_attn_r)


def _out_call(st, W_out, b_out_r, *, interpret=False):
    return pl.pallas_call(
        _out_body,
        grid=(_B,),
        in_specs=[
            pl.BlockSpec((1, _D, _NQ), lambda b: (b, 0, 0)),
            pl.BlockSpec((_D, _D), lambda b: (0, 0)),
            pl.BlockSpec((1, _D), lambda b: (0, 0)),
        ],
        out_specs=pl.BlockSpec((1, _NQ, _D), lambda b: (b, 0, 0)),
        out_shape=jax.ShapeDtypeStruct((_B, _NQ, _D), jnp.float32),
        interpret=interpret,
    )(st, W_out, b_out_r)


def kernel(query, value, W_off, b_off, W_attn, b_attn, W_val, b_val, W_out,
           b_out, spatial_shape, *, interpret=False):
    qt = query.transpose(0, 2, 1)
    vtin = value.transpose(0, 2, 1)
    W_off_r = W_off.reshape(_NH, _NP, 2, _D).transpose(2, 1, 0, 3).reshape(2 * _NP * _NH, _D)
    b_off_r = b_off.reshape(_NH, _NP, 2).transpose(2, 1, 0).reshape(2 * _NP * _NH, 1)
    W_attn_r = W_attn.reshape(_NH, _NP, _D).transpose(1, 0, 2).reshape(_NP * _NH, _D)
    b_attn_r = b_attn.reshape(_NH, _NP).transpose(1, 0).reshape(_NP * _NH, 1)
    b_val_c = b_val.reshape(_D, 1)
    b_out_r = b_out.reshape(1, _D)

    vt, idx, wgt = _prep_call(qt, vtin, W_val, b_val_c, W_off_r, b_off_r,
                              W_attn_r, b_attn_r, interpret=interpret)

    st_flat = _sc_gather(vt.reshape(_B, _D * _NQ),
                         idx.reshape(_B, _NH * _NPC * _NQ),
                         wgt.reshape(_B, _NH * _NPC * _NQ),
                         interpret=interpret)
    st = st_flat.reshape(_B, _D, _NQ)

    return _out_call(st, W_out, b_out_r, interpret=interpret)


# speedup vs baseline: 98.9458x; 1.1501x over previous
import functools

import jax
import jax.numpy as jnp
from jax import lax
from jax.experimental import pallas as pl
from jax.experimental.pallas import tpu as pltpu
from jax.experimental.pallas import tpu_sc as plsc

_B, _NQ, _D = 16, 1024, 256
_H, _W, _NH, _NP = 32, 32, 8, 4
_HD = _D // _NH
_NPC = _NP * 4
_NC, _NS = 2, 16
_NWORK = _NC * _NS
_PAIRS_PER_W = (_B * _NH) // _NWORK


def _prep_body(q_ref, v_ref, wval_ref, bval_ref, woff_ref, boff_ref,
               wattn_ref, battn_ref, vh_ref, idx_ref, wgt_ref):
    qb = q_ref[0]
    vb = v_ref[0]

    for h in range(_NH):
        wv_h = wval_ref[h * _HD:(h + 1) * _HD, :]
        vh = lax.dot_general(vb, wv_h, (((1,), (1,)), ((), ())),
                             preferred_element_type=jnp.float32)
        vh_ref[0, h] = vh + bval_ref[h]

    offr = (lax.dot_general(woff_ref[...], qb, (((1,), (1,)), ((), ())),
                            preferred_element_type=jnp.float32)
            + boff_ref[...])
    off = jnp.tanh(offr)
    awr = (lax.dot_general(wattn_ref[...], qb, (((1,), (1,)), ((), ())),
                           preferred_element_type=jnp.float32)
           + battn_ref[...])

    aws = [awr[p * _NH:(p + 1) * _NH] for p in range(_NP)]
    m = jnp.maximum(jnp.maximum(aws[0], aws[1]), jnp.maximum(aws[2], aws[3]))
    es = [jnp.exp(a - m) for a in aws]
    rs = 1.0 / (es[0] + es[1] + es[2] + es[3])

    qi = lax.broadcasted_iota(jnp.int32, (_NH, _NQ), 1)
    gx = (qi % _W).astype(jnp.float32) * (2.0 / (_W - 1)) - 1.0
    gy = (qi // _W).astype(jnp.float32) * (2.0 / (_H - 1)) - 1.0

    for p in range(_NP):
        offx = off[p * _NH:(p + 1) * _NH]
        offy = off[32 + p * _NH:32 + (p + 1) * _NH]
        awn = es[p] * rs
        locx = jnp.clip(gx + 0.5 * offx, -1.0, 1.0)
        locy = jnp.clip(gy + 0.5 * offy, -1.0, 1.0)
        x = (locx + 1.0) * (_W / 2.0) - 0.5
        y = (locy + 1.0) * (_H / 2.0) - 0.5
        x0f = jnp.floor(x)
        y0f = jnp.floor(y)
        wx1 = x - x0f
        wy1 = y - y0f
        ix0 = x0f.astype(jnp.int32)
        iy0 = y0f.astype(jnp.int32)
        for c, (cy, cx) in enumerate(((0, 0), (0, 1), (1, 0), (1, 1))):
            ix = ix0 + cx
            iy = iy0 + cy
            wx = wx1 if cx else 1.0 - wx1
            wy = wy1 if cy else 1.0 - wy1
            valid = ((ix >= 0) & (ix <= _W - 1) & (iy >= 0) & (iy <= _H - 1))
            idxc = jnp.clip(iy, 0, _H - 1) * _W + jnp.clip(ix, 0, _W - 1)
            wc = wx * wy * awn * valid.astype(jnp.float32)
            pc = c * _NP + p
            idx_ref[0, :, pc, :] = idxc * _HD
            wgt_ref[0, :, pc, :] = wc


def _out_body(sh_ref, wout_ref, bout_ref, o_ref):
    acc = bout_ref[...]
    out = None
    for h in range(_NH):
        part = lax.dot_general(sh_ref[0, h], wout_ref[h],
                               (((1,), (1,)), ((), ())),
                               preferred_element_type=jnp.float32)
        out = part if out is None else out + part
    o_ref[0] = out + acc


def _sc_body(vh_hbm, idx_hbm, wgt_hbm, out_hbm, table, idxs, wgts, outv):
    wid = lax.axis_index("c") * _NS + lax.axis_index("s")

    def pair_body(k, carry):
        e = wid * _PAIRS_PER_W + k
        b = e // _NH
        h = e - b * _NH
        toff = pl.multiple_of(h * (_NQ * _HD), 8)
        ioff = pl.multiple_of(h * (_NPC * _NQ), 8)
        pltpu.sync_copy(vh_hbm.at[b, pl.ds(toff, _NQ * _HD)], table)
        pltpu.sync_copy(idx_hbm.at[b, pl.ds(ioff, _NPC * _NQ)], idxs)
        pltpu.sync_copy(wgt_hbm.at[b, pl.ds(ioff, _NPC * _NQ)], wgts)

        def q_body(qb, qcarry):
            q0 = pl.multiple_of(qb * 16, 16)
            rows_v = [idxs[pl.ds(pl.multiple_of(pc * _NQ, 16) + q0, 16)]
                      for pc in range(_NPC)]
            w_v = [wgts[pl.ds(pl.multiple_of(pc * _NQ, 16) + q0, 16)]
                   for pc in range(_NPC)]
            for u in range(16):
                acc0 = jnp.zeros((16,), jnp.float32)
                acc1 = jnp.zeros((16,), jnp.float32)
                for pc in range(_NPC):
                    base = pl.multiple_of(rows_v[pc][u], 8)
                    w = w_v[pc][u]
                    g0 = table[pl.ds(base, 16)]
                    g1 = table[pl.ds(base + 16, 16)]
                    acc0 = acc0 + w * g0
                    acc1 = acc1 + w * g1
                o0 = pl.multiple_of((q0 + u) * _HD, 8)
                outv[pl.ds(o0, 16)] = acc0
                outv[pl.ds(o0 + 16, 16)] = acc1
            return qcarry

        lax.fori_loop(0, _NQ // 16, q_body, 0)
        pltpu.sync_copy(outv, out_hbm.at[b, pl.ds(toff, _NQ * _HD)])
        return carry

    lax.fori_loop(0, _PAIRS_PER_W, pair_body, 0)


def _sc_gather(vh_flat, idx_flat, wgt_flat):
    mesh = plsc.VectorSubcoreMesh(core_axis_name="c", subcore_axis_name="s",
                                  num_cores=_NC, num_subcores=_NS)
    return pl.kernel(
        _sc_body,
        out_type=jax.ShapeDtypeStruct((_B, _NH * _NQ * _HD), jnp.float32),
        mesh=mesh,
        scratch_types=[
            pltpu.VMEM((_NQ * _HD,), jnp.float32),
            pltpu.VMEM((_NPC * _NQ,), jnp.int32),
            pltpu.VMEM((_NPC * _NQ,), jnp.float32),
            pltpu.VMEM((_NQ * _HD,), jnp.float32),
        ],
        compiler_params=pltpu.CompilerParams(needs_layout_passes=False),
    )(vh_flat, idx_flat, wgt_flat)


def _prep_call(query, value, W_val, b_val_r, W_off_r, b_off_r, W_attn_r,
               b_attn_r, *, interpret=False):
    full = lambda shape: pl.BlockSpec(shape, lambda b: (0,) * len(shape))
    return pl.pallas_call(
        _prep_body,
        grid=(_B,),
        in_specs=[
            pl.BlockSpec((1, _NQ, _D), lambda b: (b, 0, 0)),
            pl.BlockSpec((1, _NQ, _D), lambda b: (b, 0, 0)),
            full((_D, _D)),
            full((_NH, 1, _HD)),
            full((2 * _NP * _NH, _D)),
            full((2 * _NP * _NH, 1)),
            full((_NP * _NH, _D)),
            full((_NP * _NH, 1)),
        ],
        out_specs=[
            pl.BlockSpec((1, _NH, _NQ, _HD), lambda b: (b, 0, 0, 0)),
            pl.BlockSpec((1, _NH, _NPC, _NQ), lambda b: (b, 0, 0, 0)),
            pl.BlockSpec((1, _NH, _NPC, _NQ), lambda b: (b, 0, 0, 0)),
        ],
        out_shape=[
            jax.ShapeDtypeStruct((_B, _NH, _NQ, _HD), jnp.float32),
            jax.ShapeDtypeStruct((_B, _NH, _NPC, _NQ), jnp.int32),
            jax.ShapeDtypeStruct((_B, _NH, _NPC, _NQ), jnp.float32),
        ],
        interpret=interpret,
    )(query, value, W_val, b_val_r, W_off_r, b_off_r, W_attn_r, b_attn_r)


def _out_call(sh, W_out_r, b_out_r, *, interpret=False):
    return pl.pallas_call(
        _out_body,
        grid=(_B,),
        in_specs=[
            pl.BlockSpec((1, _NH, _NQ, _HD), lambda b: (b, 0, 0, 0)),
            pl.BlockSpec((_NH, _D, _HD), lambda b: (0, 0, 0)),
            pl.BlockSpec((1, _D), lambda b: (0, 0)),
        ],
        out_specs=pl.BlockSpec((1, _NQ, _D), lambda b: (b, 0, 0)),
        out_shape=jax.ShapeDtypeStruct((_B, _NQ, _D), jnp.float32),
        interpret=interpret,
    )(sh, W_out_r, b_out_r)


def kernel(query, value, W_off, b_off, W_attn, b_attn, W_val, b_val, W_out,
           b_out, spatial_shape, *, interpret=False):
    W_off_r = W_off.reshape(_NH, _NP, 2, _D).transpose(2, 1, 0, 3).reshape(2 * _NP * _NH, _D)
    b_off_r = b_off.reshape(_NH, _NP, 2).transpose(2, 1, 0).reshape(2 * _NP * _NH, 1)
    W_attn_r = W_attn.reshape(_NH, _NP, _D).transpose(1, 0, 2).reshape(_NP * _NH, _D)
    b_attn_r = b_attn.reshape(_NH, _NP).transpose(1, 0).reshape(_NP * _NH, 1)
    b_val_r = b_val.reshape(_NH, 1, _HD)
    W_out_r = W_out.reshape(_D, _NH, _HD).transpose(1, 0, 2)
    b_out_r = b_out.reshape(1, _D)

    vh, idx, wgt = _prep_call(query, value, W_val, b_val_r, W_off_r, b_off_r,
                              W_attn_r, b_attn_r, interpret=interpret)

    sh_flat = _sc_gather(vh.reshape(_B, _NH * _NQ * _HD),
                         idx.reshape(_B, _NH * _NPC * _NQ),
                         wgt.reshape(_B, _NH * _NPC * _NQ))
    sh = sh_flat.reshape(_B, _NH, _NQ, _HD)

    return _out_call(sh, W_out_r, b_out_r, interpret=interpret)
